# baseline (speedup 1.0000x reference)
import jax
import jax.numpy as jnp
from jax.experimental import pallas as pl

N_ROWS = 16384
D = 64
K = 1024
BETA = 0.25
BLOCK = 4096
GRID = N_ROWS // BLOCK


def _vq_block(x_ref, cb_ref, fsqr_ref, csqr_ref, iotaf_ref,
              zq_ref, idx_ref, loss_ref):
    i = pl.program_id(0)
    x = x_ref[...]
    cb = cb_ref[...]
    csqr = csqr_ref[...]
    iotaf = iotaf_ref[...]

    scores = jax.lax.dot_general(
        x, cb, dimension_numbers=(((1,), (1,)), ((), ())),
        preferred_element_type=jnp.float32)
    fsqr = fsqr_ref[...]
    dist = fsqr - 2.0 * scores + csqr

    min_d = jnp.min(dist, axis=1)

    sel = jnp.where(dist == min_d[:, None], iotaf, float(K))
    idx_f = jnp.min(sel, axis=1)
    idx = idx_f.astype(jnp.int32)

    onehot = (iotaf == idx_f[:, None]).astype(jnp.float32)
    q = jax.lax.dot_general(
        onehot, cb, dimension_numbers=(((1,), (0,)), ((), ())),
        preferred_element_type=jnp.float32)

    zq_ref[...] = x + (q - x)
    idx_ref[...] = idx.reshape(1, 1, BLOCK)

    part = jnp.sum(min_d).reshape(1, 1)

    @pl.when(i == 0)
    def _init():
        loss_ref[...] = jnp.zeros_like(loss_ref)

    loss_ref[...] += part


def kernel(inputs, codebook):
    x = inputs.reshape(N_ROWS, D)
    fsqr = jnp.sum(x ** 2, axis=-1, keepdims=True)
    csqr = jnp.sum(codebook ** 2, axis=-1, keepdims=True).T
    iotaf = jnp.arange(K, dtype=jnp.float32).reshape(1, K)

    zq, idx3, loss_sum = pl.pallas_call(
        _vq_block,
        grid=(GRID,),
        in_specs=[
            pl.BlockSpec((BLOCK, D), lambda i: (i, 0)),
            pl.BlockSpec((K, D), lambda i: (0, 0)),
            pl.BlockSpec((BLOCK, 1), lambda i: (i, 0)),
            pl.BlockSpec((1, K), lambda i: (0, 0)),
            pl.BlockSpec((1, K), lambda i: (0, 0)),
        ],
        out_specs=[
            pl.BlockSpec((BLOCK, D), lambda i: (i, 0)),
            pl.BlockSpec((1, 1, BLOCK), lambda i: (i, 0, 0)),
            pl.BlockSpec((1, 1), lambda i: (0, 0)),
        ],
        out_shape=[
            jax.ShapeDtypeStruct((N_ROWS, D), jnp.float32),
            jax.ShapeDtypeStruct((GRID, 1, BLOCK), jnp.int32),
            jax.ShapeDtypeStruct((1, 1), jnp.float32),
        ],
    )(x, codebook, fsqr, csqr, iotaf)

    loss = loss_sum[0, 0] * ((1.0 + BETA) / (N_ROWS * D))
    z_q = zq.reshape(inputs.shape)
    encoding_indices = idx3.reshape(inputs.shape[:-1])
    return (loss, z_q, encoding_indices)


# speedup vs baseline: 1.0055x; 1.0055x over previous
import jax
import jax.numpy as jnp
from jax.experimental import pallas as pl

N_ROWS = 16384
D = 64
K = 1024
BETA = 0.25
BLOCK = 2048
GRID = N_ROWS // BLOCK


def _vq_block(x_ref, cb_ref, fsqr_ref, csqr_ref, iotaf_ref,
              zq_ref, idx_ref, loss_ref):
    i = pl.program_id(0)
    x = x_ref[...]
    cb = cb_ref[...]
    csqr = csqr_ref[...]
    iotaf = iotaf_ref[...]

    scores = jax.lax.dot_general(
        x, cb, dimension_numbers=(((1,), (1,)), ((), ())),
        preferred_element_type=jnp.float32)
    fsqr = fsqr_ref[...]
    dist = fsqr - 2.0 * scores + csqr

    min_d = jnp.min(dist, axis=1)

    sel = jnp.where(dist == min_d[:, None], iotaf, float(K))
    idx_f = jnp.min(sel, axis=1)
    idx = idx_f.astype(jnp.int32)

    onehot = (iotaf == idx_f[:, None]).astype(jnp.float32)
    q = jax.lax.dot_general(
        onehot, cb, dimension_numbers=(((1,), (0,)), ((), ())),
        preferred_element_type=jnp.float32)

    zq_ref[...] = x + (q - x)
    idx_ref[...] = idx.reshape(1, 1, BLOCK)

    part = jnp.sum(min_d).reshape(1, 1)

    @pl.when(i == 0)
    def _init():
        loss_ref[...] = jnp.zeros_like(loss_ref)

    loss_ref[...] += part


def kernel(inputs, codebook):
    x = inputs.reshape(N_ROWS, D)
    fsqr = jnp.sum(x ** 2, axis=-1, keepdims=True)
    csqr = jnp.sum(codebook ** 2, axis=-1, keepdims=True).T
    iotaf = jnp.arange(K, dtype=jnp.float32).reshape(1, K)

    zq, idx3, loss_sum = pl.pallas_call(
        _vq_block,
        grid=(GRID,),
        in_specs=[
            pl.BlockSpec((BLOCK, D), lambda i: (i, 0)),
            pl.BlockSpec((K, D), lambda i: (0, 0)),
            pl.BlockSpec((BLOCK, 1), lambda i: (i, 0)),
            pl.BlockSpec((1, K), lambda i: (0, 0)),
            pl.BlockSpec((1, K), lambda i: (0, 0)),
        ],
        out_specs=[
            pl.BlockSpec((BLOCK, D), lambda i: (i, 0)),
            pl.BlockSpec((1, 1, BLOCK), lambda i: (i, 0, 0)),
            pl.BlockSpec((1, 1), lambda i: (0, 0)),
        ],
        out_shape=[
            jax.ShapeDtypeStruct((N_ROWS, D), jnp.float32),
            jax.ShapeDtypeStruct((GRID, 1, BLOCK), jnp.int32),
            jax.ShapeDtypeStruct((1, 1), jnp.float32),
        ],
    )(x, codebook, fsqr, csqr, iotaf)

    loss = loss_sum[0, 0] * ((1.0 + BETA) / (N_ROWS * D))
    z_q = zq.reshape(inputs.shape)
    encoding_indices = idx3.reshape(inputs.shape[:-1])
    return (loss, z_q, encoding_indices)
